# baseline (device time: 59037 ns/iter reference)
import jax
import jax.numpy as jnp
from jax import lax
from jax.experimental import pallas as pl
from jax.experimental.pallas import tpu as pltpu

N_DEV = 8
B, SQ, D = 4, 256, 1024
T = B * SQ
SC = SQ // N_DEV
HEADS, DH = 8, 128
SCALE = 0.08838834764831843


def _body(x_ref, wq_ref, wo_ref, wk_ref, wv_ref, out_ref,
          pbuf, rbuf, rs_send, rs_recv, ag_send, ag_recv):
    my = lax.axis_index("i")

    barrier = pltpu.get_barrier_semaphore()
    for j in range(1, N_DEV):
        pl.semaphore_signal(
            barrier, inc=1,
            device_id=(lax.rem(my + j, N_DEV),),
            device_id_type=pl.DeviceIdType.MESH)
    pl.semaphore_wait(barrier, N_DEV - 1)

    def attn_batch(b):
        xb_ = x_ref[b * SQ:(b + 1) * SQ, :]
        qb = jnp.dot(xb_, wq_ref[:, :], preferred_element_type=jnp.float32)
        kb = jnp.dot(xb_, wk_ref[:, :], preferred_element_type=jnp.float32)
        vb = jnp.dot(xb_, wv_ref[:, :], preferred_element_type=jnp.float32)
        heads = []
        for h in range(HEADS):
            qh = qb[:, h * DH:(h + 1) * DH]
            kh = kb[:, h * DH:(h + 1) * DH]
            vh = vb[:, h * DH:(h + 1) * DH]
            s = jnp.dot(qh, kh.T, preferred_element_type=jnp.float32) * SCALE
            m = jnp.max(s, axis=-1, keepdims=True)
            p = jnp.exp(s - m)
            l = jnp.sum(p, axis=-1, keepdims=True)
            heads.append(jnp.dot(p, vh, preferred_element_type=jnp.float32) / l)
        return jnp.concatenate(heads, axis=1)

    rs_rdmas = []
    ag_rdmas = []

    def rs_send_batch(b):
        for d in range(N_DEV):
            jj = lax.rem(my - d + N_DEV, N_DEV)
            rdma = pltpu.make_async_remote_copy(
                src_ref=pbuf.at[pl.ds(b * SQ + d * SC, SC), :],
                dst_ref=rbuf.at[b, jj],
                send_sem=rs_send.at[b, d],
                recv_sem=rs_recv.at[b, jj],
                device_id=(d,),
                device_id_type=pl.DeviceIdType.MESH,
            )
            @pl.when(jj != 0)
            def _(rdma=rdma):
                rdma.start()
            rs_rdmas.append((jj, rdma))

    def reduce_and_broadcast(b):
        for j in range(1, N_DEV):
            pltpu.make_async_remote_copy(
                src_ref=rbuf.at[b, j], dst_ref=rbuf.at[b, j],
                send_sem=rs_send.at[b, 0], recv_sem=rs_recv.at[b, j],
                device_id=(my,), device_id_type=pl.DeviceIdType.MESH,
            ).wait_recv()
        red = pbuf[pl.ds(b * SQ + my * SC, SC), :].astype(jnp.float32)
        for j in range(1, N_DEV):
            red = red + rbuf[b, j].astype(jnp.float32)
        own = pl.ds(b * SQ + my * SC, SC)
        out_ref[own, :] = red
        for j in range(1, N_DEV):
            rdma = pltpu.make_async_remote_copy(
                src_ref=out_ref.at[own, :],
                dst_ref=out_ref.at[own, :],
                send_sem=ag_send.at[b, j],
                recv_sem=ag_recv.at[b, my],
                device_id=(lax.rem(my - j + N_DEV, N_DEV),),
                device_id_type=pl.DeviceIdType.MESH,
            )
            rdma.start()
            ag_rdmas.append(rdma)

    for b in range(B):
        ab = attn_batch(b)
        pbuf[pl.ds(b * SQ, SQ), :] = jnp.dot(
            ab, wo_ref[:, :], preferred_element_type=jnp.float32
        ).astype(jnp.bfloat16)
        rs_send_batch(b)
        if b >= 1:
            reduce_and_broadcast(b - 1)
    reduce_and_broadcast(B - 1)

    for b in range(B):
        for d in range(N_DEV):
            @pl.when(d != my)
            def _(b=b, d=d):
                pltpu.make_async_remote_copy(
                    src_ref=out_ref.at[pl.ds(b * SQ + d * SC, SC), :],
                    dst_ref=out_ref.at[pl.ds(b * SQ + d * SC, SC), :],
                    send_sem=ag_send.at[b, 0], recv_sem=ag_recv.at[b, d],
                    device_id=(my,), device_id_type=pl.DeviceIdType.MESH,
                ).wait_recv()

    for jj, rdma in rs_rdmas:
        @pl.when(jj != 0)
        def _(rdma=rdma):
            rdma.wait_send()
    for rdma in ag_rdmas:
        rdma.wait_send()


def kernel(x, Wq, Wo, Wk, Wv):
    x2 = x.reshape(T, D)
    out = pl.pallas_call(
        _body,
        out_shape=jax.ShapeDtypeStruct((T, D), jnp.float32),
        in_specs=[pl.BlockSpec(memory_space=pltpu.VMEM)] * 5,
        out_specs=pl.BlockSpec(memory_space=pltpu.VMEM),
        scratch_shapes=[
            pltpu.VMEM((T, D), jnp.bfloat16),
            pltpu.VMEM((B, N_DEV, SC, D), jnp.bfloat16),
            pltpu.SemaphoreType.DMA((B, N_DEV)),
            pltpu.SemaphoreType.DMA((B, N_DEV)),
            pltpu.SemaphoreType.DMA((B, N_DEV)),
            pltpu.SemaphoreType.DMA((B, N_DEV)),
        ],
        compiler_params=pltpu.CompilerParams(collective_id=0),
    )(x2, Wq, Wo, Wk, Wv)
    return out.reshape(B, SQ, D)


# device time: 47264 ns/iter; 1.2491x vs baseline; 1.2491x over previous
import jax
import jax.numpy as jnp
from jax import lax
from jax.experimental import pallas as pl
from jax.experimental.pallas import tpu as pltpu

N_DEV = 8
B, SQ, D = 4, 256, 1024
T = B * SQ
SC = SQ // N_DEV
HEADS, DH = 8, 128
SCALE = 0.08838834764831843


def _body(x_ref, wq_ref, wo_ref, wk_ref, wv_ref, out_ref,
          pbuf, rbuf, gbuf, agbuf, rs_send, rs_recv, ag_send, ag_recv):
    my = lax.axis_index("i")

    barrier = pltpu.get_barrier_semaphore()
    for j in range(1, N_DEV):
        pl.semaphore_signal(
            barrier, inc=1,
            device_id=(lax.rem(my + j, N_DEV),),
            device_id_type=pl.DeviceIdType.MESH)

    def attn_batch(b):
        xb_ = x_ref[b * SQ:(b + 1) * SQ, :]
        qb = jnp.dot(xb_, wq_ref[:, :], preferred_element_type=jnp.float32)
        kb = jnp.dot(xb_, wk_ref[:, :], preferred_element_type=jnp.float32)
        vb = jnp.dot(xb_, wv_ref[:, :], preferred_element_type=jnp.float32)
        heads = []
        for h in range(HEADS):
            qh = qb[:, h * DH:(h + 1) * DH]
            kh = kb[:, h * DH:(h + 1) * DH]
            vh = vb[:, h * DH:(h + 1) * DH]
            s = jnp.dot(qh, kh.T, preferred_element_type=jnp.float32) * SCALE
            m = jnp.max(s, axis=-1, keepdims=True)
            p = jnp.exp(s - m)
            l = jnp.sum(p, axis=-1, keepdims=True)
            heads.append(jnp.dot(p, vh, preferred_element_type=jnp.float32) / l)
        return jnp.concatenate(heads, axis=1)

    rs_rdmas = []
    ag_rdmas = []

    def rs_send_batch(b):
        for d in range(N_DEV):
            jj = lax.rem(my - d + N_DEV, N_DEV)
            rdma = pltpu.make_async_remote_copy(
                src_ref=pbuf.at[pl.ds(b * SQ + d * SC, SC), :],
                dst_ref=rbuf.at[b, jj],
                send_sem=rs_send.at[b, d],
                recv_sem=rs_recv.at[b, jj],
                device_id=(d,),
                device_id_type=pl.DeviceIdType.MESH,
            )
            @pl.when(jj != 0)
            def _(rdma=rdma):
                rdma.start()
            rs_rdmas.append((jj, rdma))

    def reduce_and_broadcast(b):
        for j in range(1, N_DEV):
            pltpu.make_async_remote_copy(
                src_ref=rbuf.at[b, j], dst_ref=rbuf.at[b, j],
                send_sem=rs_send.at[b, 0], recv_sem=rs_recv.at[b, j],
                device_id=(my,), device_id_type=pl.DeviceIdType.MESH,
            ).wait_recv()
        own = pl.ds(b * SQ + my * SC, SC)
        red = pbuf[own, :].astype(jnp.float32)
        for j in range(1, N_DEV):
            red = red + rbuf[b, j].astype(jnp.float32)
        out_ref[own, :] = red
        gbuf[b, :, :] = red.astype(jnp.bfloat16)
        for j in range(1, N_DEV):
            rdma = pltpu.make_async_remote_copy(
                src_ref=gbuf.at[b],
                dst_ref=agbuf.at[b, j],
                send_sem=ag_send.at[b, j],
                recv_sem=ag_recv.at[b, j],
                device_id=(lax.rem(my - j + N_DEV, N_DEV),),
                device_id_type=pl.DeviceIdType.MESH,
            )
            rdma.start()
            ag_rdmas.append(rdma)

    def drain(b):
        for j in range(1, N_DEV):
            pltpu.make_async_remote_copy(
                src_ref=agbuf.at[b, j], dst_ref=agbuf.at[b, j],
                send_sem=ag_send.at[b, 0], recv_sem=ag_recv.at[b, j],
                device_id=(my,), device_id_type=pl.DeviceIdType.MESH,
            ).wait_recv()
            d = lax.rem(my + j, N_DEV)
            out_ref[pl.ds(b * SQ + d * SC, SC), :] = (
                agbuf[b, j].astype(jnp.float32))

    for b in range(B):
        ab = attn_batch(b)
        pbuf[pl.ds(b * SQ, SQ), :] = jnp.dot(
            ab, wo_ref[:, :], preferred_element_type=jnp.float32
        ).astype(jnp.bfloat16)
        if b == 0:
            pl.semaphore_wait(barrier, N_DEV - 1)
        rs_send_batch(b)
        if b >= 1:
            reduce_and_broadcast(b - 1)
    drain(0)
    drain(1)
    reduce_and_broadcast(B - 1)
    drain(2)
    drain(3)

    for jj, rdma in rs_rdmas:
        @pl.when(jj != 0)
        def _(rdma=rdma):
            rdma.wait_send()
    for rdma in ag_rdmas:
        rdma.wait_send()


def kernel(x, Wq, Wo, Wk, Wv):
    x2 = x.reshape(T, D)
    out = pl.pallas_call(
        _body,
        out_shape=jax.ShapeDtypeStruct((T, D), jnp.float32),
        in_specs=[pl.BlockSpec(memory_space=pltpu.VMEM)] * 5,
        out_specs=pl.BlockSpec(memory_space=pltpu.VMEM),
        scratch_shapes=[
            pltpu.VMEM((T, D), jnp.bfloat16),
            pltpu.VMEM((B, N_DEV, SC, D), jnp.bfloat16),
            pltpu.VMEM((B, SC, D), jnp.bfloat16),
            pltpu.VMEM((B, N_DEV, SC, D), jnp.bfloat16),
            pltpu.SemaphoreType.DMA((B, N_DEV)),
            pltpu.SemaphoreType.DMA((B, N_DEV)),
            pltpu.SemaphoreType.DMA((B, N_DEV)),
            pltpu.SemaphoreType.DMA((B, N_DEV)),
        ],
        compiler_params=pltpu.CompilerParams(collective_id=0),
    )(x2, Wq, Wo, Wk, Wv)
    return out.reshape(B, SQ, D)
